# baseline (speedup 1.0000x reference)
import functools

import jax
import jax.numpy as jnp
from jax import lax
from jax.experimental import pallas as pl
from jax.experimental.pallas import tpu as pltpu
from jax.experimental.pallas import tpu_sc as plsc

D_MODEL = 1024
D_HIDDEN = 4096
NUM_EXPERTS = 8
TOP_K = 2
N_TOK = 2048

TM = 128
MP = 4096 + NUM_EXPERTS * TM
NT = MP // TM
HC = 512
NH = D_HIDDEN // HC

SC_NC = 2
SC_NS = 16
SC_NW = SC_NC * SC_NS

NEG_INF = float("-inf")


def _gating_body(x_ref, wg_ref, bg_ref, logits_ref, idx_ref, gates_ref):
    l = jnp.dot(x_ref[...], wg_ref[...], preferred_element_type=jnp.float32)
    l = l + bg_ref[...][None, :]
    ci = lax.broadcasted_iota(jnp.int32, (N_TOK, 128), 1)
    lm = jnp.where(ci < NUM_EXPERTS, l, NEG_INF)
    m1 = jnp.max(lm, axis=1, keepdims=True)
    i1 = jnp.min(jnp.where(lm == m1, ci, 128), axis=1, keepdims=True)
    lm2 = jnp.where(ci == i1, NEG_INF, lm)
    m2 = jnp.max(lm2, axis=1, keepdims=True)
    i2 = jnp.min(jnp.where(lm2 == m2, ci, 128), axis=1, keepdims=True)
    e2 = jnp.exp(m2 - m1)
    g1 = 1.0 / (1.0 + e2)
    g2 = e2 * g1
    logits_ref[...] = l
    idx_ref[...] = jnp.where(ci == 0, i1, 0) + jnp.where(ci == 1, i2, 0)
    gates_ref[...] = jnp.where(ci == 0, g1, 0.0) + jnp.where(ci == 1, g2, 0.0)


def _gating(xr, wg_pad, bg_pad):
    return pl.pallas_call(
        _gating_body,
        out_shape=(
            jax.ShapeDtypeStruct((N_TOK, 128), jnp.float32),
            jax.ShapeDtypeStruct((N_TOK, 128), jnp.int32),
            jax.ShapeDtypeStruct((N_TOK, 128), jnp.float32),
        ),
    )(xr, wg_pad, bg_pad)


GW = MP // SC_NW
GCH = 32
GNC = GW // GCH


@functools.partial(
    pl.kernel,
    out_type=jax.ShapeDtypeStruct((MP, D_MODEL), jnp.float32),
    mesh=plsc.VectorSubcoreMesh(core_axis_name="c", subcore_axis_name="s"),
    scratch_types=[
        pltpu.VMEM((GCH,), jnp.int32),
        pltpu.VMEM((GCH, D_MODEL), jnp.float32),
        pltpu.SemaphoreType.DMA,
    ],
)
def _sc_gather(x_hbm, idx_hbm, out_hbm, idx_v, rows_v, sem):
    wid = lax.axis_index("s") * SC_NC + lax.axis_index("c")

    def chunk(c, carry):
        base = wid * GW + c * GCH
        pltpu.sync_copy(idx_hbm.at[pl.ds(base, GCH)], idx_v)
        pltpu.async_copy(x_hbm.at[idx_v], rows_v, sem).wait()
        pltpu.sync_copy(rows_v, out_hbm.at[pl.ds(base, GCH)])
        return carry

    lax.fori_loop(0, GNC, chunk, 0)


def _ffn_body(be_ref, x_ref, w1_ref, b1_ref, w2_ref, b2_ref, g_ref, out_ref):
    j = pl.program_id(1)
    h = jnp.dot(x_ref[...], w1_ref[0], preferred_element_type=jnp.float32)
    h = jnp.maximum(h + b1_ref[0], 0.0)
    acc = jnp.dot(h, w2_ref[0], preferred_element_type=jnp.float32)

    @pl.when(j == 0)
    def _():
        out_ref[...] = acc

    @pl.when(j != 0)
    def _():
        out_ref[...] = out_ref[...] + acc

    @pl.when(j == NH - 1)
    def _():
        out_ref[...] = (out_ref[...] + b2_ref[0]) * g_ref[...]


def _grouped_ffn(block_expert, xs, W1, b1, W2, b2, sgate):
    grid_spec = pltpu.PrefetchScalarGridSpec(
        num_scalar_prefetch=1,
        grid=(NT, NH),
        in_specs=[
            pl.BlockSpec((TM, D_MODEL), lambda i, j, be: (i, 0)),
            pl.BlockSpec((1, D_MODEL, HC), lambda i, j, be: (be[i], 0, j)),
            pl.BlockSpec((1, 1, HC), lambda i, j, be: (be[i], 0, j)),
            pl.BlockSpec((1, HC, D_MODEL), lambda i, j, be: (be[i], j, 0)),
            pl.BlockSpec((1, 1, D_MODEL), lambda i, j, be: (be[i], 0, 0)),
            pl.BlockSpec((TM, 1), lambda i, j, be: (i, 0)),
        ],
        out_specs=pl.BlockSpec((TM, D_MODEL), lambda i, j, be: (i, 0)),
    )
    return pl.pallas_call(
        _ffn_body,
        grid_spec=grid_spec,
        out_shape=jax.ShapeDtypeStruct((MP, D_MODEL), jnp.float32),
        compiler_params=pltpu.CompilerParams(
            dimension_semantics=("arbitrary", "arbitrary")),
    )(block_expert, xs, W1, b1.reshape(NUM_EXPERTS, 1, D_HIDDEN),
      W2, b2.reshape(NUM_EXPERTS, 1, D_MODEL), sgate)


CW = N_TOK // SC_NW
CCH = 16
CNC = CW // CCH


@functools.partial(
    pl.kernel,
    out_type=jax.ShapeDtypeStruct((N_TOK, D_MODEL), jnp.float32),
    mesh=plsc.VectorSubcoreMesh(core_axis_name="c", subcore_axis_name="s"),
    scratch_types=[
        pltpu.VMEM((CCH,), jnp.int32),
        pltpu.VMEM((CCH,), jnp.int32),
        pltpu.VMEM((CCH, D_MODEL), jnp.float32),
        pltpu.VMEM((CCH, D_MODEL), jnp.float32),
        pltpu.SemaphoreType.DMA,
    ],
)
def _sc_combine(ys_hbm, p0_hbm, p1_hbm, out_hbm, p0_v, p1_v, r0_v, r1_v, sem):
    wid = lax.axis_index("s") * SC_NC + lax.axis_index("c")

    def chunk(c, carry):
        base = wid * CW + c * CCH
        pltpu.sync_copy(p0_hbm.at[pl.ds(base, CCH)], p0_v)
        pltpu.sync_copy(p1_hbm.at[pl.ds(base, CCH)], p1_v)
        d0 = pltpu.async_copy(ys_hbm.at[p0_v], r0_v, sem)
        d1 = pltpu.async_copy(ys_hbm.at[p1_v], r1_v, sem)
        d0.wait()
        d1.wait()
        for t in range(CCH):
            def lane(k, cc, t=t):
                o = k * 16
                r0_v[t, pl.ds(o, 16)] = (
                    r0_v[t, pl.ds(o, 16)] + r1_v[t, pl.ds(o, 16)])
                return cc
            lax.fori_loop(0, D_MODEL // 16, lane, 0)
        pltpu.sync_copy(r0_v, out_hbm.at[pl.ds(base, CCH)])
        return carry

    lax.fori_loop(0, CNC, chunk, 0)


def kernel(x, Wg, bg, W1, b1, W2, b2):
    B, S, D = x.shape
    xr = x.reshape(-1, D)

    wg_pad = jnp.zeros((D_MODEL, 128), jnp.float32).at[:, :NUM_EXPERTS].set(Wg)
    bg_pad = jnp.zeros((128,), jnp.float32).at[:NUM_EXPERTS].set(bg)
    logits128, idx128, gates128 = _gating(xr, wg_pad, bg_pad)
    gating_logits = logits128[:, :NUM_EXPERTS]

    eflat = idx128[:, :TOP_K].reshape(-1)
    gflat = gates128[:, :TOP_K].reshape(-1)
    tok = jnp.arange(N_TOK * TOP_K, dtype=jnp.int32) // TOP_K
    oh = (eflat[:, None] == jnp.arange(NUM_EXPERTS, dtype=jnp.int32)[None, :])
    cum = jnp.cumsum(oh.astype(jnp.int32), axis=0)
    counts = cum[-1]
    rank = jnp.take_along_axis(cum, eflat[:, None], axis=1)[:, 0] - 1
    padded = ((counts + TM - 1) // TM) * TM
    pstart = jnp.concatenate(
        [jnp.zeros((1,), jnp.int32), jnp.cumsum(padded)[:-1].astype(jnp.int32)])
    dest = pstart[eflat] + rank
    gather_tok = jnp.zeros((MP,), jnp.int32).at[dest].set(tok)
    sgate = jnp.zeros((MP, 1), jnp.float32).at[dest, 0].set(gflat)
    tile_start = pstart // TM
    block_expert = (jnp.sum(
        jnp.arange(NT, dtype=jnp.int32)[:, None] >= tile_start[None, :],
        axis=1) - 1).astype(jnp.int32)
    p2 = dest.reshape(N_TOK, TOP_K)

    xs = _sc_gather(xr, gather_tok)
    ys = _grouped_ffn(block_expert, xs, W1, b1, W2, b2, sgate)
    out = _sc_combine(ys, p2[:, 0], p2[:, 1])

    return out.reshape(B, S, D), gating_logits


# speedup vs baseline: 1.5710x; 1.5710x over previous
import functools

import jax
import jax.numpy as jnp
from jax import lax
from jax.experimental import pallas as pl
from jax.experimental.pallas import tpu as pltpu
from jax.experimental.pallas import tpu_sc as plsc

D_MODEL = 1024
D_HIDDEN = 4096
NUM_EXPERTS = 8
TOP_K = 2
N_TOK = 2048

TM = 128
MP = 4096 + NUM_EXPERTS * TM
NT = MP // TM
HC = 1024
NH = D_HIDDEN // HC

SC_NC = 2
SC_NS = 16
SC_NW = SC_NC * SC_NS

NEG_INF = float("-inf")


def _gating_body(x_ref, wg_ref, bg_ref, logits_ref, idx_ref, gates_ref):
    l = jnp.dot(x_ref[...], wg_ref[...], preferred_element_type=jnp.float32)
    l = l + bg_ref[...][None, :]
    ci = lax.broadcasted_iota(jnp.int32, (N_TOK, 128), 1)
    lm = jnp.where(ci < NUM_EXPERTS, l, NEG_INF)
    m1 = jnp.max(lm, axis=1, keepdims=True)
    i1 = jnp.min(jnp.where(lm == m1, ci, 128), axis=1, keepdims=True)
    lm2 = jnp.where(ci == i1, NEG_INF, lm)
    m2 = jnp.max(lm2, axis=1, keepdims=True)
    i2 = jnp.min(jnp.where(lm2 == m2, ci, 128), axis=1, keepdims=True)
    e2 = jnp.exp(m2 - m1)
    g1 = 1.0 / (1.0 + e2)
    g2 = e2 * g1
    logits_ref[...] = l
    idx_ref[...] = jnp.where(ci == 0, i1, 0) + jnp.where(ci == 1, i2, 0)
    gates_ref[...] = jnp.where(ci == 0, g1, 0.0) + jnp.where(ci == 1, g2, 0.0)


def _gating(xr, wg_pad, bg_pad):
    return pl.pallas_call(
        _gating_body,
        out_shape=(
            jax.ShapeDtypeStruct((N_TOK, 128), jnp.float32),
            jax.ShapeDtypeStruct((N_TOK, 128), jnp.int32),
            jax.ShapeDtypeStruct((N_TOK, 128), jnp.float32),
        ),
    )(xr, wg_pad, bg_pad)


GW = MP // SC_NW
GCH = 32
GNC = GW // GCH


@functools.partial(
    pl.kernel,
    out_type=jax.ShapeDtypeStruct((MP, D_MODEL), jnp.float32),
    mesh=plsc.VectorSubcoreMesh(core_axis_name="c", subcore_axis_name="s"),
    scratch_types=[
        pltpu.VMEM((GCH,), jnp.int32),
        pltpu.VMEM((GCH, D_MODEL), jnp.float32),
        pltpu.SemaphoreType.DMA,
    ],
)
def _sc_gather(x_hbm, idx_hbm, out_hbm, idx_v, rows_v, sem):
    wid = lax.axis_index("s") * SC_NC + lax.axis_index("c")

    def chunk(c, carry):
        base = wid * GW + c * GCH
        pltpu.sync_copy(idx_hbm.at[pl.ds(base, GCH)], idx_v)
        pltpu.async_copy(x_hbm.at[idx_v], rows_v, sem).wait()
        pltpu.sync_copy(rows_v, out_hbm.at[pl.ds(base, GCH)])
        return carry

    lax.fori_loop(0, GNC, chunk, 0)


def _ffn_body(be_ref, x_ref, w1_ref, b1_ref, w2_ref, b2_ref, g_ref, out_ref,
              acc_ref):
    j = pl.program_id(0)
    i = pl.program_id(1)
    h = jnp.dot(x_ref[...], w1_ref[0], preferred_element_type=jnp.float32)
    h = jnp.maximum(h + b1_ref[0], 0.0)
    part = jnp.dot(h, w2_ref[0], preferred_element_type=jnp.float32)
    row = i * TM

    @pl.when(j == 0)
    def _():
        acc_ref[pl.ds(row, TM), :] = part

    @pl.when(j != 0)
    def _():
        acc_ref[pl.ds(row, TM), :] = acc_ref[pl.ds(row, TM), :] + part

    @pl.when(j == NH - 1)
    def _():
        out_ref[...] = (acc_ref[pl.ds(row, TM), :] + b2_ref[0]) * g_ref[...]


def _grouped_ffn(block_expert, xs, W1, b1, W2, b2, sgate):
    grid_spec = pltpu.PrefetchScalarGridSpec(
        num_scalar_prefetch=1,
        grid=(NH, NT),
        in_specs=[
            pl.BlockSpec((TM, D_MODEL), lambda j, i, be: (i, 0)),
            pl.BlockSpec((1, D_MODEL, HC), lambda j, i, be: (be[i], 0, j)),
            pl.BlockSpec((1, 1, HC), lambda j, i, be: (be[i], 0, j)),
            pl.BlockSpec((1, HC, D_MODEL), lambda j, i, be: (be[i], j, 0)),
            pl.BlockSpec((1, 1, D_MODEL), lambda j, i, be: (be[i], 0, 0)),
            pl.BlockSpec((TM, 1), lambda j, i, be: (i, 0)),
        ],
        out_specs=pl.BlockSpec(
            (TM, D_MODEL),
            lambda j, i, be: (jnp.where(j == NH - 1, i, 0), 0)),
        scratch_shapes=[pltpu.VMEM((MP, D_MODEL), jnp.float32)],
    )
    return pl.pallas_call(
        _ffn_body,
        grid_spec=grid_spec,
        out_shape=jax.ShapeDtypeStruct((MP, D_MODEL), jnp.float32),
        compiler_params=pltpu.CompilerParams(
            dimension_semantics=("arbitrary", "arbitrary")),
    )(block_expert, xs, W1, b1.reshape(NUM_EXPERTS, 1, D_HIDDEN),
      W2, b2.reshape(NUM_EXPERTS, 1, D_MODEL), sgate)


CW = N_TOK // SC_NW
CCH = 16
CNC = CW // CCH


@functools.partial(
    pl.kernel,
    out_type=jax.ShapeDtypeStruct((N_TOK, D_MODEL), jnp.float32),
    mesh=plsc.VectorSubcoreMesh(core_axis_name="c", subcore_axis_name="s"),
    scratch_types=[
        pltpu.VMEM((CCH,), jnp.int32),
        pltpu.VMEM((CCH,), jnp.int32),
        pltpu.VMEM((CCH, D_MODEL), jnp.float32),
        pltpu.VMEM((CCH, D_MODEL), jnp.float32),
        pltpu.SemaphoreType.DMA,
    ],
)
def _sc_combine(ys_hbm, p0_hbm, p1_hbm, out_hbm, p0_v, p1_v, r0_v, r1_v, sem):
    wid = lax.axis_index("s") * SC_NC + lax.axis_index("c")

    def chunk(c, carry):
        base = wid * CW + c * CCH
        pltpu.sync_copy(p0_hbm.at[pl.ds(base, CCH)], p0_v)
        pltpu.sync_copy(p1_hbm.at[pl.ds(base, CCH)], p1_v)
        d0 = pltpu.async_copy(ys_hbm.at[p0_v], r0_v, sem)
        d1 = pltpu.async_copy(ys_hbm.at[p1_v], r1_v, sem)
        d0.wait()
        d1.wait()
        for t in range(CCH):
            def lane(k, cc, t=t):
                o = k * 16
                r0_v[t, pl.ds(o, 16)] = (
                    r0_v[t, pl.ds(o, 16)] + r1_v[t, pl.ds(o, 16)])
                return cc
            lax.fori_loop(0, D_MODEL // 16, lane, 0)
        pltpu.sync_copy(r0_v, out_hbm.at[pl.ds(base, CCH)])
        return carry

    lax.fori_loop(0, CNC, chunk, 0)


def kernel(x, Wg, bg, W1, b1, W2, b2):
    B, S, D = x.shape
    xr = x.reshape(-1, D)

    wg_pad = jnp.zeros((D_MODEL, 128), jnp.float32).at[:, :NUM_EXPERTS].set(Wg)
    bg_pad = jnp.zeros((128,), jnp.float32).at[:NUM_EXPERTS].set(bg)
    logits128, idx128, gates128 = _gating(xr, wg_pad, bg_pad)
    gating_logits = logits128[:, :NUM_EXPERTS]

    eflat = idx128[:, :TOP_K].reshape(-1)
    gflat = gates128[:, :TOP_K].reshape(-1)
    tok = jnp.arange(N_TOK * TOP_K, dtype=jnp.int32) // TOP_K
    oh = (eflat[:, None] == jnp.arange(NUM_EXPERTS, dtype=jnp.int32)[None, :])
    cum = jnp.cumsum(oh.astype(jnp.int32), axis=0)
    counts = cum[-1]
    rank = jnp.take_along_axis(cum, eflat[:, None], axis=1)[:, 0] - 1
    padded = ((counts + TM - 1) // TM) * TM
    pstart = jnp.concatenate(
        [jnp.zeros((1,), jnp.int32), jnp.cumsum(padded)[:-1].astype(jnp.int32)])
    dest = pstart[eflat] + rank
    gather_tok = jnp.zeros((MP,), jnp.int32).at[dest].set(tok)
    sgate = jnp.zeros((MP, 1), jnp.float32).at[dest, 0].set(gflat)
    tile_start = pstart // TM
    block_expert = (jnp.sum(
        jnp.arange(NT, dtype=jnp.int32)[:, None] >= tile_start[None, :],
        axis=1) - 1).astype(jnp.int32)
    p2 = dest.reshape(N_TOK, TOP_K)

    xs = _sc_gather(xr, gather_tok)
    ys = _grouped_ffn(block_expert, xs, W1, b1, W2, b2, sgate)
    out = _sc_combine(ys, p2[:, 0], p2[:, 1])

    return out.reshape(B, S, D), gating_logits


# speedup vs baseline: 1.5971x; 1.0166x over previous
import functools

import jax
import jax.numpy as jnp
from jax import lax
from jax.experimental import pallas as pl
from jax.experimental.pallas import tpu as pltpu
from jax.experimental.pallas import tpu_sc as plsc

D_MODEL = 1024
D_HIDDEN = 4096
NUM_EXPERTS = 8
TOP_K = 2
N_TOK = 2048

TM = 128
MP = 4096 + NUM_EXPERTS * TM
NT = MP // TM
HC = 1024
NH = D_HIDDEN // HC

SC_NC = 2
SC_NS = 16
SC_NW = SC_NC * SC_NS

NEG_INF = float("-inf")


def _gating_body(x_ref, wg_ref, bg_ref, logits_ref, idx_ref, gates_ref):
    l = jnp.dot(x_ref[...], wg_ref[...], preferred_element_type=jnp.float32)
    l = l + bg_ref[...][None, :]
    ci = lax.broadcasted_iota(jnp.int32, (N_TOK, 128), 1)
    lm = jnp.where(ci < NUM_EXPERTS, l, NEG_INF)
    m1 = jnp.max(lm, axis=1, keepdims=True)
    i1 = jnp.min(jnp.where(lm == m1, ci, 128), axis=1, keepdims=True)
    lm2 = jnp.where(ci == i1, NEG_INF, lm)
    m2 = jnp.max(lm2, axis=1, keepdims=True)
    i2 = jnp.min(jnp.where(lm2 == m2, ci, 128), axis=1, keepdims=True)
    e2 = jnp.exp(m2 - m1)
    g1 = 1.0 / (1.0 + e2)
    g2 = e2 * g1
    logits_ref[...] = l
    idx_ref[...] = jnp.where(ci == 0, i1, 0) + jnp.where(ci == 1, i2, 0)
    gates_ref[...] = jnp.where(ci == 0, g1, 0.0) + jnp.where(ci == 1, g2, 0.0)


def _gating(xr, wg_pad, bg_pad):
    return pl.pallas_call(
        _gating_body,
        out_shape=(
            jax.ShapeDtypeStruct((N_TOK, 128), jnp.float32),
            jax.ShapeDtypeStruct((N_TOK, 128), jnp.int32),
            jax.ShapeDtypeStruct((N_TOK, 128), jnp.float32),
        ),
    )(xr, wg_pad, bg_pad)


GW = MP // SC_NW
GCH = 40
GNC = GW // GCH


@functools.partial(
    pl.kernel,
    out_type=jax.ShapeDtypeStruct((MP, D_MODEL), jnp.float32),
    mesh=plsc.VectorSubcoreMesh(core_axis_name="c", subcore_axis_name="s"),
    scratch_types=[
        pltpu.VMEM((2, GCH), jnp.int32),
        pltpu.VMEM((GCH, D_MODEL), jnp.float32),
        pltpu.VMEM((GCH, D_MODEL), jnp.float32),
        pltpu.SemaphoreType.DMA,
        pltpu.SemaphoreType.DMA,
        pltpu.SemaphoreType.DMA,
        pltpu.SemaphoreType.DMA,
    ],
)
def _sc_gather(x_hbm, idx_hbm, out_hbm, idx_v, rows_a, rows_b, si, sa, sb, sw):
    wid = lax.axis_index("s") * SC_NC + lax.axis_index("c")
    base = wid * GW
    rows = (rows_a, rows_b)
    gsem = (sa, sb)
    pltpu.sync_copy(idx_hbm.at[pl.ds(base, GCH)], idx_v.at[0])
    gathers = [pltpu.async_copy(x_hbm.at[idx_v.at[0]], rows_a, sa)]
    wbs = [None, None]
    for c in range(GNC):
        cur = c % 2
        nxt = (c + 1) % 2
        if c + 1 < GNC:
            pltpu.sync_copy(idx_hbm.at[pl.ds(base + (c + 1) * GCH, GCH)],
                            idx_v.at[nxt])
            if wbs[nxt] is not None:
                wbs[nxt].wait()
                wbs[nxt] = None
            gathers.append(pltpu.async_copy(
                x_hbm.at[idx_v.at[nxt]], rows[nxt], gsem[nxt]))
        gathers[c].wait()
        wbs[cur] = pltpu.async_copy(
            rows[cur], out_hbm.at[pl.ds(base + c * GCH, GCH)], sw)
    for wb in wbs:
        if wb is not None:
            wb.wait()


def _ffn_body(be_ref, x_ref, w1_ref, b1_ref, w2_ref, b2_ref, g_ref, out_ref,
              acc_ref):
    j = pl.program_id(0)
    i = pl.program_id(1)
    h = jnp.dot(x_ref[...], w1_ref[0], preferred_element_type=jnp.float32)
    h = jnp.maximum(h + b1_ref[0], 0.0)
    part = jnp.dot(h, w2_ref[0], preferred_element_type=jnp.float32)
    row = i * TM

    @pl.when(j == 0)
    def _():
        acc_ref[pl.ds(row, TM), :] = part

    @pl.when(j != 0)
    def _():
        acc_ref[pl.ds(row, TM), :] = acc_ref[pl.ds(row, TM), :] + part

    @pl.when(j == NH - 1)
    def _():
        out_ref[...] = (acc_ref[pl.ds(row, TM), :] + b2_ref[0]) * g_ref[...]


def _grouped_ffn(block_expert, xs, W1, b1, W2, b2, sgate):
    grid_spec = pltpu.PrefetchScalarGridSpec(
        num_scalar_prefetch=1,
        grid=(NH, NT),
        in_specs=[
            pl.BlockSpec((TM, D_MODEL), lambda j, i, be: (i, 0)),
            pl.BlockSpec((1, D_MODEL, HC), lambda j, i, be: (be[i], 0, j)),
            pl.BlockSpec((1, 1, HC), lambda j, i, be: (be[i], 0, j)),
            pl.BlockSpec((1, HC, D_MODEL), lambda j, i, be: (be[i], j, 0)),
            pl.BlockSpec((1, 1, D_MODEL), lambda j, i, be: (be[i], 0, 0)),
            pl.BlockSpec((TM, 1), lambda j, i, be: (i, 0)),
        ],
        out_specs=pl.BlockSpec(
            (TM, D_MODEL),
            lambda j, i, be: (jnp.where(j == NH - 1, i, 0), 0)),
        scratch_shapes=[pltpu.VMEM((MP, D_MODEL), jnp.float32)],
    )
    return pl.pallas_call(
        _ffn_body,
        grid_spec=grid_spec,
        out_shape=jax.ShapeDtypeStruct((MP, D_MODEL), jnp.float32),
        compiler_params=pltpu.CompilerParams(
            dimension_semantics=("arbitrary", "arbitrary")),
    )(block_expert, xs, W1, b1.reshape(NUM_EXPERTS, 1, D_HIDDEN),
      W2, b2.reshape(NUM_EXPERTS, 1, D_MODEL), sgate)


CW = N_TOK // SC_NW
CCH = 16
CNC = CW // CCH


@functools.partial(
    pl.kernel,
    out_type=jax.ShapeDtypeStruct((N_TOK, D_MODEL), jnp.float32),
    mesh=plsc.VectorSubcoreMesh(core_axis_name="c", subcore_axis_name="s"),
    scratch_types=[
        pltpu.VMEM((2, CCH), jnp.int32),
        pltpu.VMEM((2, CCH), jnp.int32),
        pltpu.VMEM((CCH, D_MODEL), jnp.float32),
        pltpu.VMEM((CCH, D_MODEL), jnp.float32),
        pltpu.VMEM((CCH, D_MODEL), jnp.float32),
        pltpu.VMEM((CCH, D_MODEL), jnp.float32),
        pltpu.SemaphoreType.DMA,
        pltpu.SemaphoreType.DMA,
        pltpu.SemaphoreType.DMA,
    ],
)
def _sc_combine(ys_hbm, p0_hbm, p1_hbm, out_hbm, p0_v, p1_v,
                r0_a, r0_b, r1_a, r1_b, sa, sb, sw):
    wid = lax.axis_index("s") * SC_NC + lax.axis_index("c")
    base = wid * CW
    r0 = (r0_a, r0_b)
    r1 = (r1_a, r1_b)
    gsem = (sa, sb)
    pltpu.sync_copy(p0_hbm.at[pl.ds(base, CCH)], p0_v.at[0])
    pltpu.sync_copy(p1_hbm.at[pl.ds(base, CCH)], p1_v.at[0])
    gathers = [(pltpu.async_copy(ys_hbm.at[p0_v.at[0]], r0_a, sa),
                pltpu.async_copy(ys_hbm.at[p1_v.at[0]], r1_a, sa))]
    wbs = [None, None]
    for c in range(CNC):
        cur = c % 2
        nxt = (c + 1) % 2
        if c + 1 < CNC:
            pltpu.sync_copy(p0_hbm.at[pl.ds(base + (c + 1) * CCH, CCH)],
                            p0_v.at[nxt])
            pltpu.sync_copy(p1_hbm.at[pl.ds(base + (c + 1) * CCH, CCH)],
                            p1_v.at[nxt])
            if wbs[nxt] is not None:
                wbs[nxt].wait()
                wbs[nxt] = None
            gathers.append(
                (pltpu.async_copy(ys_hbm.at[p0_v.at[nxt]], r0[nxt], gsem[nxt]),
                 pltpu.async_copy(ys_hbm.at[p1_v.at[nxt]], r1[nxt], gsem[nxt])))
        gathers[c][0].wait()
        gathers[c][1].wait()
        for t in range(CCH):
            def lane(k, cc, t=t, cur=cur):
                o = k * 16
                r0[cur][t, pl.ds(o, 16)] = (
                    r0[cur][t, pl.ds(o, 16)] + r1[cur][t, pl.ds(o, 16)])
                return cc
            lax.fori_loop(0, D_MODEL // 16, lane, 0)
        wbs[cur] = pltpu.async_copy(
            r0[cur], out_hbm.at[pl.ds(base + c * CCH, CCH)], sw)
    for wb in wbs:
        if wb is not None:
            wb.wait()


def kernel(x, Wg, bg, W1, b1, W2, b2):
    B, S, D = x.shape
    xr = x.reshape(-1, D)

    wg_pad = jnp.zeros((D_MODEL, 128), jnp.float32).at[:, :NUM_EXPERTS].set(Wg)
    bg_pad = jnp.zeros((128,), jnp.float32).at[:NUM_EXPERTS].set(bg)
    logits128, idx128, gates128 = _gating(xr, wg_pad, bg_pad)
    gating_logits = logits128[:, :NUM_EXPERTS]

    eflat = idx128[:, :TOP_K].reshape(-1)
    gflat = gates128[:, :TOP_K].reshape(-1)
    tok = jnp.arange(N_TOK * TOP_K, dtype=jnp.int32) // TOP_K
    oh = (eflat[:, None] == jnp.arange(NUM_EXPERTS, dtype=jnp.int32)[None, :])
    cum = jnp.cumsum(oh.astype(jnp.int32), axis=0)
    counts = cum[-1]
    rank = jnp.take_along_axis(cum, eflat[:, None], axis=1)[:, 0] - 1
    padded = ((counts + TM - 1) // TM) * TM
    pstart = jnp.concatenate(
        [jnp.zeros((1,), jnp.int32), jnp.cumsum(padded)[:-1].astype(jnp.int32)])
    dest = pstart[eflat] + rank
    gather_tok = jnp.zeros((MP,), jnp.int32).at[dest].set(tok)
    sgate = jnp.zeros((MP, 1), jnp.float32).at[dest, 0].set(gflat)
    tile_start = pstart // TM
    block_expert = (jnp.sum(
        jnp.arange(NT, dtype=jnp.int32)[:, None] >= tile_start[None, :],
        axis=1) - 1).astype(jnp.int32)
    p2 = dest.reshape(N_TOK, TOP_K)

    xs = _sc_gather(xr, gather_tok)
    ys = _grouped_ffn(block_expert, xs, W1, b1, W2, b2, sgate)
    out = _sc_combine(ys, p2[:, 0], p2[:, 1])

    return out.reshape(B, S, D), gating_logits
